# baseline (speedup 1.0000x reference)
import jax
import jax.numpy as jnp
from jax.experimental import pallas as pl
from jax.experimental.pallas import tpu as pltpu

TOP_K = 6
ROUTE_SCALE = 2.5
BLOCK_T = 512


def _gate_kernel(x_ref, w_ref, b_ref, wout_ref, iout_ref, s_ref):
    i = pl.program_id(0)
    x = x_ref[...]
    w = w_ref[...]
    n_experts = w.shape[0]
    scores = jax.lax.dot_general(
        x, w, (((1,), (1,)), ((), ())),
        preferred_element_type=jnp.float32,
        precision=jax.lax.Precision.DEFAULT)
    scores = jnp.sqrt(jax.nn.softplus(scores))
    s_ref[jax.lax.rem(i, 2)] = scores

    sc = s_ref[jax.lax.rem(i + 1, 2)]
    b = sc + b_ref[...]
    colsf = jax.lax.broadcasted_iota(
        jnp.int32, b.shape, 1).astype(jnp.float32)
    nf = jnp.float32(n_experts)
    neg_inf = jnp.float32(-jnp.inf)
    ws, idxs = [], []
    for _ in range(TOP_K):
        m = jnp.max(b, axis=1, keepdims=True)
        idxf = jnp.min(jnp.where(b == m, colsf, nf), axis=1)
        onehot = colsf == idxf[:, None]
        ws.append(jnp.sum(jnp.where(onehot, sc, 0.0), axis=1))
        idxs.append(idxf)
        b = jnp.where(onehot, neg_inf, b)
    w_stack = jnp.stack(ws, axis=1)
    i_stack = jnp.stack(idxs, axis=1).astype(jnp.int32)
    w_stack = w_stack / jnp.sum(w_stack, axis=1, keepdims=True) * ROUTE_SCALE
    wout_ref[...] = w_stack
    iout_ref[...] = i_stack


def kernel(x, weight, bias):
    tokens, dim = x.shape
    n_experts = weight.shape[0]
    bias2d = bias.reshape(1, n_experts)
    nblocks = tokens // BLOCK_T
    grid = (nblocks + 1,)
    wout, iout = pl.pallas_call(
        _gate_kernel,
        grid=grid,
        in_specs=[
            pl.BlockSpec((BLOCK_T, dim),
                         lambda i: (jnp.minimum(i, nblocks - 1), 0)),
            pl.BlockSpec((n_experts, dim), lambda i: (0, 0)),
            pl.BlockSpec((1, n_experts), lambda i: (0, 0)),
        ],
        out_specs=[
            pl.BlockSpec((BLOCK_T, TOP_K), lambda i: (jnp.maximum(i - 1, 0), 0)),
            pl.BlockSpec((BLOCK_T, TOP_K), lambda i: (jnp.maximum(i - 1, 0), 0)),
        ],
        out_shape=[
            jax.ShapeDtypeStruct((tokens, TOP_K), jnp.float32),
            jax.ShapeDtypeStruct((tokens, TOP_K), jnp.int32),
        ],
        scratch_shapes=[pltpu.VMEM((2, BLOCK_T, n_experts), jnp.float32)],
    )(x, weight, bias2d)
    return (wout, iout)


# speedup vs baseline: 1.1451x; 1.1451x over previous
import jax
import jax.numpy as jnp
from jax.experimental import pallas as pl
from jax.experimental.pallas import tpu as pltpu

TOP_K = 6
ROUTE_SCALE = 2.5
BLOCK_T = 512


def _gate_kernel(x_ref, w_ref, b_ref, wout_ref, iout_ref, s0_ref, s1_ref):
    i = pl.program_id(0)
    n_experts = w_ref.shape[0]

    def matmul_and_tail(store_ref, load_ref):
        x = x_ref[...]
        w = w_ref[...]
        scores = jax.lax.dot_general(
            x, w, (((1,), (1,)), ((), ())),
            preferred_element_type=jnp.float32,
            precision=jax.lax.Precision.DEFAULT)
        scores = jnp.sqrt(jax.nn.softplus(scores))
        store_ref[...] = scores

        sc = load_ref[...]
        b = sc + b_ref[...]
        colsf = jax.lax.broadcasted_iota(
            jnp.int32, b.shape, 1).astype(jnp.float32)
        nf = jnp.float32(n_experts)
        neg_inf = jnp.float32(-jnp.inf)
        ws, idxs = [], []
        for _ in range(TOP_K):
            m = jnp.max(b, axis=1, keepdims=True)
            idxf = jnp.min(jnp.where(b == m, colsf, nf), axis=1)
            onehot = colsf == idxf[:, None]
            ws.append(jnp.sum(jnp.where(onehot, sc, 0.0), axis=1))
            idxs.append(idxf)
            b = jnp.where(onehot, neg_inf, b)
        w_stack = jnp.stack(ws, axis=1)
        i_stack = jnp.stack(idxs, axis=1).astype(jnp.int32)
        w_stack = (w_stack / jnp.sum(w_stack, axis=1, keepdims=True)
                   * ROUTE_SCALE)
        wout_ref[...] = w_stack
        iout_ref[...] = i_stack

    @pl.when(jax.lax.rem(i, 2) == 0)
    def _():
        matmul_and_tail(s0_ref, s1_ref)

    @pl.when(jax.lax.rem(i, 2) == 1)
    def _():
        matmul_and_tail(s1_ref, s0_ref)


def kernel(x, weight, bias):
    tokens, dim = x.shape
    n_experts = weight.shape[0]
    bias2d = bias.reshape(1, n_experts)
    nblocks = tokens // BLOCK_T
    grid = (nblocks + 1,)
    wout, iout = pl.pallas_call(
        _gate_kernel,
        grid=grid,
        in_specs=[
            pl.BlockSpec((BLOCK_T, dim),
                         lambda i: (jnp.minimum(i, nblocks - 1), 0)),
            pl.BlockSpec((n_experts, dim), lambda i: (0, 0)),
            pl.BlockSpec((1, n_experts), lambda i: (0, 0)),
        ],
        out_specs=[
            pl.BlockSpec((BLOCK_T, TOP_K), lambda i: (jnp.maximum(i - 1, 0), 0)),
            pl.BlockSpec((BLOCK_T, TOP_K), lambda i: (jnp.maximum(i - 1, 0), 0)),
        ],
        out_shape=[
            jax.ShapeDtypeStruct((tokens, TOP_K), jnp.float32),
            jax.ShapeDtypeStruct((tokens, TOP_K), jnp.int32),
        ],
        scratch_shapes=[
            pltpu.VMEM((BLOCK_T, n_experts), jnp.float32),
            pltpu.VMEM((BLOCK_T, n_experts), jnp.float32),
        ],
    )(x, weight, bias2d)
    return (wout, iout)


# speedup vs baseline: 1.1582x; 1.0114x over previous
import jax
import jax.numpy as jnp
from jax.experimental import pallas as pl
from jax.experimental.pallas import tpu as pltpu

TOP_K = 6
ROUTE_SCALE = 2.5
BLOCK_T = 512
CHUNK_T = 64


def _gate_kernel(x_ref, w_ref, b_ref, wout_ref, iout_ref, s0_ref, s1_ref):
    i = pl.program_id(0)
    n_experts = w_ref.shape[0]

    def matmul_and_tail(store_ref, load_ref):
        x = x_ref[...]
        w = w_ref[...]
        scores = jax.lax.dot_general(
            x, w, (((1,), (1,)), ((), ())),
            preferred_element_type=jnp.float32,
            precision=jax.lax.Precision.DEFAULT)
        scores = jnp.sqrt(jax.nn.softplus(scores))
        store_ref[...] = scores

        biasv = b_ref[...]
        colsf = jax.lax.broadcasted_iota(
            jnp.int32, (1, n_experts), 1).astype(jnp.float32)
        nf = jnp.float32(n_experts)
        neg_inf = jnp.float32(-jnp.inf)
        for c in range(BLOCK_T // CHUNK_T):
            lo = c * CHUNK_T
            sc = load_ref[lo:lo + CHUNK_T, :]
            b = sc + biasv
            ws, idxs = [], []
            for _ in range(TOP_K):
                m = jnp.max(b, axis=1, keepdims=True)
                idxf = jnp.min(jnp.where(b == m, colsf, nf), axis=1)
                onehot = colsf == idxf[:, None]
                ws.append(jnp.sum(jnp.where(onehot, sc, 0.0), axis=1))
                idxs.append(idxf)
                b = jnp.where(onehot, neg_inf, b)
            w_stack = jnp.stack(ws, axis=1)
            i_stack = jnp.stack(idxs, axis=1).astype(jnp.int32)
            w_stack = (w_stack / jnp.sum(w_stack, axis=1, keepdims=True)
                       * ROUTE_SCALE)
            wout_ref[lo:lo + CHUNK_T, :] = w_stack
            iout_ref[lo:lo + CHUNK_T, :] = i_stack

    @pl.when(jax.lax.rem(i, 2) == 0)
    def _():
        matmul_and_tail(s0_ref, s1_ref)

    @pl.when(jax.lax.rem(i, 2) == 1)
    def _():
        matmul_and_tail(s1_ref, s0_ref)


def kernel(x, weight, bias):
    tokens, dim = x.shape
    n_experts = weight.shape[0]
    bias2d = bias.reshape(1, n_experts)
    nblocks = tokens // BLOCK_T
    grid = (nblocks + 1,)
    wout, iout = pl.pallas_call(
        _gate_kernel,
        grid=grid,
        in_specs=[
            pl.BlockSpec((BLOCK_T, dim),
                         lambda i: (jnp.minimum(i, nblocks - 1), 0)),
            pl.BlockSpec((n_experts, dim), lambda i: (0, 0)),
            pl.BlockSpec((1, n_experts), lambda i: (0, 0)),
        ],
        out_specs=[
            pl.BlockSpec((BLOCK_T, TOP_K), lambda i: (jnp.maximum(i - 1, 0), 0)),
            pl.BlockSpec((BLOCK_T, TOP_K), lambda i: (jnp.maximum(i - 1, 0), 0)),
        ],
        out_shape=[
            jax.ShapeDtypeStruct((tokens, TOP_K), jnp.float32),
            jax.ShapeDtypeStruct((tokens, TOP_K), jnp.int32),
        ],
        scratch_shapes=[
            pltpu.VMEM((BLOCK_T, n_experts), jnp.float32),
            pltpu.VMEM((BLOCK_T, n_experts), jnp.float32),
        ],
    )(x, weight, bias2d)
    return (wout, iout)
